# baseline (speedup 1.0000x reference)
import functools
import jax
import jax.numpy as jnp
from jax import lax
from jax.experimental import pallas as pl
from jax.experimental.pallas import tpu as pltpu
from jax.experimental.pallas import tpu_sc as plsc

B, S, D = 4, 4096, 1024
NC, NS = 2, 16
NW = NC * NS
SEQ_PER_W = S // NW
R = 16
NCHUNK = SEQ_PER_W // R
CH = R * D


def sc_kernel_make():
    mesh = plsc.VectorSubcoreMesh(core_axis_name="c", subcore_axis_name="s")

    @functools.partial(
        pl.kernel,
        mesh=mesh,
        out_type=jax.ShapeDtypeStruct((B, S * D), jnp.float32),
        scratch_types=[
            pltpu.VMEM((CH,), jnp.float32),
            pltpu.VMEM((CH,), jnp.float32),
            pltpu.VMEM((CH,), jnp.float32),
        ],
    )
    def k(x_hbm, pe_hbm, out_hbm, pe_v, xa_v, xb_v):
        wid = lax.axis_index("s") * NC + lax.axis_index("c")
        seq0 = wid * SEQ_PER_W
        for c in range(NCHUNK):
            off = seq0 * D + c * CH
            pltpu.sync_copy(pe_hbm.at[pl.ds(off, CH)], pe_v)
            for b in range(B):
                xv = xa_v if b % 2 == 0 else xb_v
                pltpu.sync_copy(x_hbm.at[b, pl.ds(off, CH)], xv)

                def add16(i, _):
                    sl = pl.ds(i * 16, 16)
                    xv[sl] = xv[sl] + pe_v[sl]
                    return 0

                lax.fori_loop(0, CH // 16, add16, 0)
                pltpu.sync_copy(xv, out_hbm.at[b, pl.ds(off, CH)])

    return k


def kernel(x, pos_embedding):
    xf = x.reshape(B, S * D)
    pef = pos_embedding.reshape(-1)
    out = sc_kernel_make()(xf, pef)
    return out.reshape(B, S, D)


# speedup vs baseline: 1.8046x; 1.8046x over previous
import functools
import jax
import jax.numpy as jnp
from jax import lax
from jax.experimental import pallas as pl
from jax.experimental.pallas import tpu as pltpu
from jax.experimental.pallas import tpu_sc as plsc

B, S, D = 4, 4096, 1024
NC, NS = 2, 16
NW = NC * NS
SEQ_PER_W = S // NW
R = 16
NCHUNK = SEQ_PER_W // R
CH = R * D
NXB = 4
NSTEP = NCHUNK * B


def _make():
    mesh = plsc.VectorSubcoreMesh(core_axis_name="c", subcore_axis_name="s")

    @functools.partial(
        pl.kernel,
        mesh=mesh,
        out_type=jax.ShapeDtypeStruct((B, S * D), jnp.float32),
        scratch_types=(
            [pltpu.VMEM((CH,), jnp.float32) for _ in range(2)]
            + [pltpu.VMEM((CH,), jnp.float32) for _ in range(NXB)]
            + [pltpu.SemaphoreType.DMA for _ in range(2 + 2 * NXB)]
        ),
    )
    def k(x_hbm, pe_hbm, out_hbm, pe0, pe1, x0, x1, x2, x3,
          ps0, ps1, xs0, xs1, xs2, xs3, os0, os1, os2, os3):
        pe_v = [pe0, pe1]
        x_v = [x0, x1, x2, x3]
        pe_sem = [ps0, ps1]
        x_sem = [xs0, xs1, xs2, xs3]
        o_sem = [os0, os1, os2, os3]

        wid = lax.axis_index("s") * NC + lax.axis_index("c")
        seq0 = wid * SEQ_PER_W

        def off(c):
            return seq0 * D + c * CH

        def issue_pe(c):
            return pltpu.async_copy(
                pe_hbm.at[pl.ds(off(c), CH)], pe_v[c % 2], pe_sem[c % 2])

        def issue_x(j):
            c, b = j // B, j % B
            return pltpu.async_copy(
                x_hbm.at[b, pl.ds(off(c), CH)], x_v[j % NXB], x_sem[j % NXB])

        def issue_out(j):
            c, b = j // B, j % B
            return pltpu.async_copy(
                x_v[j % NXB], out_hbm.at[b, pl.ds(off(c), CH)], o_sem[j % NXB])

        pe_h = [None] * NCHUNK
        x_h = [None] * NSTEP
        o_h = [None] * NSTEP

        pe_h[0] = issue_pe(0)
        pe_h[1] = issue_pe(1)
        x_h[0] = issue_x(0)
        x_h[1] = issue_x(1)

        for j in range(NSTEP):
            c, b = j // B, j % B
            xb = x_v[j % NXB]
            pb = pe_v[c % 2]
            if b == 0:
                pe_h[c].wait()
            x_h[j].wait()

            def add_body(i, _):
                base = i * 128
                for u in range(8):
                    sl = pl.ds(base + u * 16, 16)
                    xb[sl] = xb[sl] + pb[sl]
                return 0

            lax.fori_loop(0, CH // 128, add_body, 0, unroll=False)
            o_h[j] = issue_out(j)

            if j + 2 < NSTEP:
                if j - 2 >= 0:
                    o_h[j - 2].wait()
                x_h[j + 2] = issue_x(j + 2)
            if b == B - 1 and c + 2 < NCHUNK:
                pe_h[c + 2] = issue_pe(c + 2)

        o_h[NSTEP - 2].wait()
        o_h[NSTEP - 1].wait()

    return k


def kernel(x, pos_embedding):
    xf = x.reshape(B, S * D)
    pef = pos_embedding.reshape(-1)
    out = _make()(xf, pef)
    return out.reshape(B, S, D)
